# baseline (speedup 1.0000x reference)
import functools

import jax
import jax.numpy as jnp
from jax import lax
from jax.experimental import pallas as pl
from jax.experimental.pallas import tpu as pltpu, tpu_sc as plsc

BATCH = 16384
NCLASS = 360

_info = plsc.get_sparse_core_info()
_NC, _NS, _L = _info.num_cores, _info.num_subcores, _info.num_lanes
_NW = _NC * _NS
_ROWS_PER_W = BATCH // _NW
_SB_ROWS = 64
_NSB = _ROWS_PER_W // _SB_ROWS
_SUB = _SB_ROWS // _L
_NACC = 4
_CHUNK = NCLASS // _NACC
_SB_WORDS = _SB_ROWS * NCLASS
_PAD = _CHUNK + 8


def _tec_body(inputs_hbm, degs_hbm, out_hbm, buf0, buf1, degs_v, out_v, sem0, sem1):
    wid = lax.axis_index("s") * _NC + lax.axis_index("c")
    base = wid * _ROWS_PER_W

    pltpu.sync_copy(degs_hbm, degs_v)

    lanes = lax.iota(jnp.int32, _L)

    bufs = (buf0, buf1)
    sems = (sem0, sem1)

    def start_copy(sb):
        k = sb & 1
        return pltpu.async_copy(
            inputs_hbm.at[pl.ds(base + sb * _SB_ROWS, _SB_ROWS), :],
            bufs[k],
            sems[k],
        )

    pending = start_copy(0)
    for sb in range(_NSB):
        buf = bufs[sb & 1]
        pending.wait()
        if sb + 1 < _NSB:
            pending = start_copy(sb + 1)

        for b in range(_SUB):
            rows = lanes + b * _L

            def col_step(t, carry, _rows=rows, _buf=buf):
                ms, bis, tvec = carry
                new_ms, new_bis = [], []
                for a in range(_NACC):
                    x = plsc.load_gather(_buf, [_rows, tvec + a * _CHUNK])
                    gt = x > ms[a]
                    new_ms.append(jnp.where(gt, x, ms[a]))
                    new_bis.append(jnp.where(gt, tvec, bis[a]))
                return tuple(new_ms), tuple(new_bis), tvec + 1

            m0 = tuple(jnp.full((_L,), -jnp.inf, jnp.float32) for _ in range(_NACC))
            b0 = tuple(jnp.zeros((_L,), jnp.int32) for _ in range(_NACC))
            t0 = jnp.zeros((_L,), jnp.int32)
            ms, bis, _ = lax.fori_loop(0, _CHUNK, col_step, (m0, b0, t0))

            m, bi = ms[0], bis[0]
            for a in range(1, _NACC):
                gt = ms[a] > m
                m = jnp.where(gt, ms[a], m)
                bi = jnp.where(gt, bis[a] + a * _CHUNK, bi)

            d = plsc.load_gather(degs_v, [bi])
            out_v[pl.ds(sb * _SB_ROWS + b * _L, _L)] = d

    pltpu.sync_copy(out_v, out_hbm.at[pl.ds(base, _ROWS_PER_W)])


@jax.jit
def kernel(inputs, degs):
    mesh = plsc.VectorSubcoreMesh(core_axis_name="c", subcore_axis_name="s")
    run = functools.partial(
        pl.kernel,
        mesh=mesh,
        out_type=jax.ShapeDtypeStruct((BATCH,), jnp.float32),
        compiler_params=pltpu.CompilerParams(
            use_tc_tiling_on_sc=True, needs_layout_passes=False
        ),
        scratch_types=[
            pltpu.VMEM((_SB_ROWS, NCLASS), jnp.float32),
            pltpu.VMEM((_SB_ROWS, NCLASS), jnp.float32),
            pltpu.VMEM((NCLASS,), jnp.float32),
            pltpu.VMEM((_ROWS_PER_W,), jnp.float32),
            pltpu.SemaphoreType.DMA,
            pltpu.SemaphoreType.DMA,
        ],
    )(_tec_body)
    return run(inputs, degs)


# speedup vs baseline: 1.5159x; 1.5159x over previous
import functools

import jax
import jax.numpy as jnp
from jax import lax
from jax.experimental import pallas as pl
from jax.experimental.pallas import tpu as pltpu, tpu_sc as plsc

BATCH = 16384
NCLASS = 360

_info = plsc.get_sparse_core_info()
_NC, _NS, _L = _info.num_cores, _info.num_subcores, _info.num_lanes
_NW = _NC * _NS
_ROWS_PER_W = BATCH // _NW
_SB_ROWS = 64
_NSB = _ROWS_PER_W // _SB_ROWS
_SUB = _SB_ROWS // _L
_NCH = 8
_NIT = NCLASS // _NCH
_SB_WORDS = _SB_ROWS * NCLASS
_WIN = (_L - 1) * NCLASS + _NCH


def _tec_body(inputs_hbm, degs_hbm, out_hbm, buf0, buf1, degs_v, out_v, sem0, sem1):
    wid = lax.axis_index("s") * _NC + lax.axis_index("c")
    base = wid * _ROWS_PER_W

    pltpu.sync_copy(degs_hbm, degs_v)

    lane_addr = lax.iota(jnp.int32, _L) * NCLASS
    chain_idx = tuple(lane_addr + j for j in range(_NCH))

    bufs = (buf0, buf1)
    sems = (sem0, sem1)

    def start_copy(sb):
        k = sb & 1
        return pltpu.async_copy(
            inputs_hbm.at[pl.ds((base + sb * _SB_ROWS) * NCLASS, _SB_WORDS)],
            bufs[k],
            sems[k],
        )

    pending = start_copy(0)
    for sb in range(_NSB):
        buf = bufs[sb & 1]
        pending.wait()
        if sb + 1 < _NSB:
            pending = start_copy(sb + 1)

        for b in range(_SUB):
            sub_off = b * _L * NCLASS

            def col_step(i, carry, _buf=buf, _sub=sub_off):
                ms, bis = carry
                win = _buf.at[pl.ds(_sub + i * _NCH, _WIN)]
                new_ms, new_bis = [], []
                for j in range(_NCH):
                    x = plsc.load_gather(win, [chain_idx[j]])
                    gt = x > ms[j]
                    new_ms.append(jnp.where(gt, x, ms[j]))
                    new_bis.append(jnp.where(gt, i, bis[j]))
                return tuple(new_ms), tuple(new_bis)

            m0 = tuple(jnp.full((_L,), -jnp.inf, jnp.float32) for _ in range(_NCH))
            b0 = tuple(jnp.zeros((_L,), jnp.int32) for _ in range(_NCH))
            ms, bis = lax.fori_loop(0, _NIT, col_step, (m0, b0))

            m, col = ms[0], bis[0] * _NCH
            for j in range(1, _NCH):
                cj = bis[j] * _NCH + j
                take = (ms[j] > m) | ((ms[j] == m) & (cj < col))
                m = jnp.where(take, ms[j], m)
                col = jnp.where(take, cj, col)

            d = plsc.load_gather(degs_v, [col])
            out_v[pl.ds(sb * _SB_ROWS + b * _L, _L)] = d

    pltpu.sync_copy(out_v, out_hbm.at[pl.ds(base, _ROWS_PER_W)])


@jax.jit
def kernel(inputs, degs):
    mesh = plsc.VectorSubcoreMesh(core_axis_name="c", subcore_axis_name="s")
    run = functools.partial(
        pl.kernel,
        mesh=mesh,
        out_type=jax.ShapeDtypeStruct((BATCH,), jnp.float32),
        compiler_params=pltpu.CompilerParams(
            use_tc_tiling_on_sc=True, needs_layout_passes=False
        ),
        scratch_types=[
            pltpu.VMEM((_SB_WORDS,), jnp.float32),
            pltpu.VMEM((_SB_WORDS,), jnp.float32),
            pltpu.VMEM((NCLASS,), jnp.float32),
            pltpu.VMEM((_ROWS_PER_W,), jnp.float32),
            pltpu.SemaphoreType.DMA,
            pltpu.SemaphoreType.DMA,
        ],
    )(_tec_body)
    return run(inputs.reshape(-1), degs)


# speedup vs baseline: 1.5194x; 1.0023x over previous
import functools

import jax
import jax.numpy as jnp
from jax import lax
from jax.experimental import pallas as pl
from jax.experimental.pallas import tpu as pltpu, tpu_sc as plsc

BATCH = 16384
NCLASS = 360

_info = plsc.get_sparse_core_info()
_NC, _NS, _L = _info.num_cores, _info.num_subcores, _info.num_lanes
_NW = _NC * _NS
_ROWS_PER_W = BATCH // _NW
_SB_ROWS = 64
_NSB = _ROWS_PER_W // _SB_ROWS
_SUB = _SB_ROWS // _L
_NCH = 8
_NIT = NCLASS // _NCH
_SB_WORDS = _SB_ROWS * NCLASS
_WIN = (_L - 1) * NCLASS + _NCH


def _tec_body(inputs_hbm, degs_hbm, out_hbm, buf0, buf1, degs_v, out_v, sem0, sem1):
    wid = lax.axis_index("s") * _NC + lax.axis_index("c")
    base = wid * _ROWS_PER_W

    pltpu.sync_copy(degs_hbm, degs_v)

    lanes = lax.iota(jnp.int32, _L)
    lane_addr = lanes * NCLASS
    res = tuple((lanes + j) % _NCH for j in range(_NCH))
    chain_idx = tuple(lane_addr + res[j] for j in range(_NCH))

    bufs = (buf0, buf1)
    sems = (sem0, sem1)

    def start_copy(sb):
        k = sb & 1
        return pltpu.async_copy(
            inputs_hbm.at[pl.ds((base + sb * _SB_ROWS) * NCLASS, _SB_WORDS)],
            bufs[k],
            sems[k],
        )

    pending = start_copy(0)
    for sb in range(_NSB):
        buf = bufs[sb & 1]
        pending.wait()
        if sb + 1 < _NSB:
            pending = start_copy(sb + 1)

        for b in range(_SUB):
            sub_off = b * _L * NCLASS

            def col_step(i, carry, _buf=buf, _sub=sub_off):
                ms, bis = carry
                win = _buf.at[pl.ds(_sub + i * _NCH, _WIN)]
                new_ms, new_bis = [], []
                for j in range(_NCH):
                    x = plsc.load_gather(win, [chain_idx[j]])
                    gt = x > ms[j]
                    new_ms.append(jnp.where(gt, x, ms[j]))
                    new_bis.append(jnp.where(gt, i, bis[j]))
                return tuple(new_ms), tuple(new_bis)

            m0 = tuple(jnp.full((_L,), -jnp.inf, jnp.float32) for _ in range(_NCH))
            b0 = tuple(jnp.zeros((_L,), jnp.int32) for _ in range(_NCH))
            ms, bis = lax.fori_loop(0, _NIT, col_step, (m0, b0))

            m, col = ms[0], bis[0] * _NCH + res[0]
            for j in range(1, _NCH):
                cj = bis[j] * _NCH + res[j]
                take = (ms[j] > m) | ((ms[j] == m) & (cj < col))
                m = jnp.where(take, ms[j], m)
                col = jnp.where(take, cj, col)

            d = plsc.load_gather(degs_v, [col])
            out_v[pl.ds(sb * _SB_ROWS + b * _L, _L)] = d

    pltpu.sync_copy(out_v, out_hbm.at[pl.ds(base, _ROWS_PER_W)])


@jax.jit
def kernel(inputs, degs):
    mesh = plsc.VectorSubcoreMesh(core_axis_name="c", subcore_axis_name="s")
    run = functools.partial(
        pl.kernel,
        mesh=mesh,
        out_type=jax.ShapeDtypeStruct((BATCH,), jnp.float32),
        compiler_params=pltpu.CompilerParams(
            use_tc_tiling_on_sc=True, needs_layout_passes=False
        ),
        scratch_types=[
            pltpu.VMEM((_SB_WORDS,), jnp.float32),
            pltpu.VMEM((_SB_WORDS,), jnp.float32),
            pltpu.VMEM((NCLASS,), jnp.float32),
            pltpu.VMEM((_ROWS_PER_W,), jnp.float32),
            pltpu.SemaphoreType.DMA,
            pltpu.SemaphoreType.DMA,
        ],
    )(_tec_body)
    return run(inputs.reshape(-1), degs)


# speedup vs baseline: 2.0361x; 1.3401x over previous
import functools

import jax
import jax.numpy as jnp
from jax import lax
from jax.experimental import pallas as pl
from jax.experimental.pallas import tpu as pltpu, tpu_sc as plsc

BATCH = 16384
NCLASS = 360

_info = plsc.get_sparse_core_info()
_NC, _NS, _L = _info.num_cores, _info.num_subcores, _info.num_lanes
_NW = _NC * _NS
_ROWS_PER_W = BATCH // _NW
_SB_ROWS = 64
_NSB = _ROWS_PER_W // _SB_ROWS
_SUB = _SB_ROWS // _L
_NCH = 8
_NIT = NCLASS // _NCH
_SB_WORDS = _SB_ROWS * NCLASS
_WIN = (_L - 1) * NCLASS + _NCH


def _tec_body(inputs_hbm, degs_hbm, out_hbm, buf0, buf1, degs_v, out_v, sem0, sem1):
    wid = lax.axis_index("s") * _NC + lax.axis_index("c")
    base = wid * _ROWS_PER_W

    pltpu.sync_copy(degs_hbm, degs_v)

    lanes = lax.iota(jnp.int32, _L)
    res = tuple((lanes + j) % _NCH for j in range(_NCH))

    bufs = (buf0, buf1)
    sems = (sem0, sem1)

    def start_copy(sb):
        k = sb & 1
        return pltpu.async_copy(
            inputs_hbm.at[pl.ds(base + sb * _SB_ROWS, _SB_ROWS)],
            bufs[k],
            sems[k],
        )

    pending = start_copy(0)
    for sb in range(_NSB):
        buf = bufs[sb & 1]
        pending.wait()
        if sb + 1 < _NSB:
            pending = start_copy(sb + 1)

        for b in range(_SUB):
            row_lo = b * _L

            def col_step(i, carry, _buf=buf, _row=row_lo):
                ms, bis = carry
                win = _buf.at[pl.ds(_row, _L)]
                cbase = jnp.full((_L,), i * _NCH, jnp.int32)
                new_ms, new_bis = [], []
                for j in range(_NCH):
                    x = plsc.load_gather(win, [lanes, cbase + res[j]])
                    gt = x > ms[j]
                    new_ms.append(jnp.where(gt, x, ms[j]))
                    new_bis.append(jnp.where(gt, i, bis[j]))
                return tuple(new_ms), tuple(new_bis)

            m0 = tuple(jnp.full((_L,), -jnp.inf, jnp.float32) for _ in range(_NCH))
            b0 = tuple(jnp.zeros((_L,), jnp.int32) for _ in range(_NCH))
            ms, bis = lax.fori_loop(0, _NIT, col_step, (m0, b0))

            m, col = ms[0], bis[0] * _NCH + res[0]
            for j in range(1, _NCH):
                cj = bis[j] * _NCH + res[j]
                take = (ms[j] > m) | ((ms[j] == m) & (cj < col))
                m = jnp.where(take, ms[j], m)
                col = jnp.where(take, cj, col)

            d = plsc.load_gather(degs_v, [col])
            out_v[pl.ds(sb * _SB_ROWS + b * _L, _L)] = d

    pltpu.sync_copy(out_v, out_hbm.at[pl.ds(base, _ROWS_PER_W)])


@jax.jit
def kernel(inputs, degs):
    mesh = plsc.VectorSubcoreMesh(core_axis_name="c", subcore_axis_name="s")
    run = functools.partial(
        pl.kernel,
        mesh=mesh,
        out_type=jax.ShapeDtypeStruct((BATCH,), jnp.float32),
        compiler_params=pltpu.CompilerParams(
            use_tc_tiling_on_sc=True, needs_layout_passes=False
        ),
        scratch_types=[
            pltpu.VMEM((_SB_ROWS, NCLASS), jnp.float32),
            pltpu.VMEM((_SB_ROWS, NCLASS), jnp.float32),
            pltpu.VMEM((NCLASS,), jnp.float32),
            pltpu.VMEM((_ROWS_PER_W,), jnp.float32),
            pltpu.SemaphoreType.DMA,
            pltpu.SemaphoreType.DMA,
        ],
    )(_tec_body)
    return run(inputs, degs)
